# baseline (speedup 1.0000x reference)
import functools

import jax
import jax.numpy as jnp
from jax import lax
from jax.experimental import pallas as pl
from jax.experimental.pallas import tpu as pltpu
from jax.experimental.pallas import tpu_sc as plsc

N = 10000
F_IN = 128
H = 64
G = 128
E = 320000

NN = 2 * N
NC, NS = 2, 16
NW = NC * NS
RPT = 1280
NP = NS * RPT
EE = 2 * E
CH = 128
EPW = 20480
EP = NW * EPW
NCH = EPW // CH
NPH = 4
CPP = NCH // NPH

@functools.lru_cache(maxsize=None)
def _sc_mesh():
    return plsc.VectorSubcoreMesh(core_axis_name="c", subcore_axis_name="s",
                                  num_cores=NC, num_subcores=NS)


def _sc_degree_body(dst_hbm, zeros_hbm, ones_hbm, out_hbm,
                    accum, dsts, ones_v, ssem):
    c = lax.axis_index("c")
    s = lax.axis_index("s")
    wid = s * NC + c
    pltpu.sync_copy(zeros_hbm, accum.at[pl.ds(s * RPT, RPT)])
    pltpu.sync_copy(ones_hbm, ones_v)
    pltpu.sync_copy(dst_hbm.at[wid], dsts)
    plsc.subcore_barrier()

    for ph in range(NPH):
        def grp(i, carry):
            base = 8 * i
            for b in range(8):
                pltpu.async_copy(ones_v, accum.at[dsts.at[ph, base + b]], ssem,
                                 add=True)
            for b in range(8):
                pltpu.make_async_copy(ones_v, accum.at[dsts.at[ph, base + b]],
                                      ssem).wait()
            return carry

        lax.fori_loop(0, CPP // 8, grp, 0)
    plsc.subcore_barrier()
    pltpu.sync_copy(accum.at[pl.ds(s * RPT, RPT)], out_hbm.at[c, pl.ds(s * RPT, RPT)])


@functools.lru_cache(maxsize=None)
def _sc_degree_kernel():
    return pl.kernel(
        _sc_degree_body,
        out_type=jax.ShapeDtypeStruct((NC, NP), jnp.float32),
        mesh=_sc_mesh(),
        scratch_types=[
            pltpu.VMEM_SHARED((NP,), jnp.float32),
            pltpu.VMEM((NPH, CPP, CH), jnp.int32),
            pltpu.VMEM((CH,), jnp.float32),
            pltpu.SemaphoreType.DMA,
        ],
    )


def _sc_degree(dst, z1, ones):
    return _sc_degree_kernel()(dst, z1, ones)


def _sc_scatter_body(g_hbm, src_hbm, dst_hbm, zrow_hbm, out_hbm,
                     accum, srcs, dsts, rows, gsem, ssem):
    c = lax.axis_index("c")
    s = lax.axis_index("s")
    wid = s * NC + c
    pltpu.sync_copy(zrow_hbm, accum.at[pl.ds(s * RPT, RPT)])
    plsc.subcore_barrier()

    for ph in range(NPH):
        pltpu.sync_copy(src_hbm.at[wid, ph], srcs)
        pltpu.sync_copy(dst_hbm.at[wid, ph], dsts)
        pltpu.async_copy(g_hbm.at[srcs.at[0]], rows.at[0], gsem.at[0])
        pltpu.async_copy(g_hbm.at[srcs.at[1]], rows.at[1], gsem.at[1])

        def quad(i, carry):
            for b in range(4):
                cc = 4 * i + b
                bn = (b + 2) % 4

                @pl.when(cc >= 2)
                def _():
                    pltpu.make_async_copy(rows.at[bn], accum.at[dsts.at[cc - 2]],
                                          ssem.at[bn]).wait()

                @pl.when(cc + 2 < CPP)
                def _():
                    pltpu.async_copy(g_hbm.at[srcs.at[cc + 2]], rows.at[bn],
                                     gsem.at[bn])

                pltpu.make_async_copy(g_hbm.at[srcs.at[cc]], rows.at[b],
                                      gsem.at[b]).wait()
                pltpu.async_copy(rows.at[b], accum.at[dsts.at[cc]],
                                 ssem.at[b], add=True)
            return carry

        lax.fori_loop(0, CPP // 4, quad, 0)
        pltpu.make_async_copy(rows.at[2], accum.at[dsts.at[CPP - 2]],
                              ssem.at[2]).wait()
        pltpu.make_async_copy(rows.at[3], accum.at[dsts.at[CPP - 1]],
                              ssem.at[3]).wait()
    plsc.subcore_barrier()
    pltpu.sync_copy(accum.at[pl.ds(s * RPT, RPT)], out_hbm.at[c, pl.ds(s * RPT, RPT)])


@functools.lru_cache(maxsize=None)
def _sc_scatter_kernel():
    return pl.kernel(
        _sc_scatter_body,
        out_type=jax.ShapeDtypeStruct((NC, NP, H), jnp.float32),
        mesh=_sc_mesh(),
        scratch_types=[
            pltpu.VMEM_SHARED((NP, H), jnp.float32),
            pltpu.VMEM((CPP, CH), jnp.int32),
            pltpu.VMEM((CPP, CH), jnp.int32),
            pltpu.VMEM((4, CH, H), jnp.float32),
            pltpu.SemaphoreType.DMA((4,)),
            pltpu.SemaphoreType.DMA((4,)),
        ],
        compiler_params=pltpu.CompilerParams(use_tc_tiling_on_sc=False),
    )


def _sc_scatter(g, src, dst, zrow):
    return _sc_scatter_kernel()(g, src, dst, zrow)


_BLK = 1280
_NBLK = NP // _BLK


def _k1_body(x_ref, w_ref, dp_ref, g_ref, dv_ref):
    d = dp_ref[0] + dp_ref[1] + 1.0
    dv = lax.rsqrt(d)
    h = jnp.dot(x_ref[...], w_ref[...], preferred_element_type=jnp.float32)
    g_ref[...] = h * dv
    dv_ref[...] = dv


def _k1(x, w1, degp):
    return pl.pallas_call(
        _k1_body,
        grid=(_NBLK,),
        in_specs=[
            pl.BlockSpec((_BLK, F_IN), lambda i: (i, 0)),
            pl.BlockSpec((F_IN, H), lambda i: (0, 0)),
            pl.BlockSpec((NC, _BLK, 1), lambda i: (0, i, 0)),
        ],
        out_specs=[
            pl.BlockSpec((_BLK, H), lambda i: (i, 0)),
            pl.BlockSpec((_BLK, 1), lambda i: (i, 0)),
        ],
        out_shape=[
            jax.ShapeDtypeStruct((NP, H), jnp.float32),
            jax.ShapeDtypeStruct((NP, 1), jnp.float32),
        ],
    )(x, w1, degp)


def _k2_body(p_ref, g_ref, dv_ref, b_ref, w_ref, o_ref):
    s = p_ref[0] + p_ref[1] + g_ref[...]
    a = jnp.maximum(s * dv_ref[...] + b_ref[...], 0.0)
    o_ref[...] = jnp.dot(a, w_ref[...], preferred_element_type=jnp.float32) * dv_ref[...]


def _k2(p, g1, dv, b1, w2):
    return pl.pallas_call(
        _k2_body,
        grid=(_NBLK,),
        in_specs=[
            pl.BlockSpec((NC, _BLK, H), lambda i: (0, i, 0)),
            pl.BlockSpec((_BLK, H), lambda i: (i, 0)),
            pl.BlockSpec((_BLK, 1), lambda i: (i, 0)),
            pl.BlockSpec((1, H), lambda i: (0, 0)),
            pl.BlockSpec((H, H), lambda i: (0, 0)),
        ],
        out_specs=pl.BlockSpec((_BLK, H), lambda i: (i, 0)),
        out_shape=jax.ShapeDtypeStruct((NP, H), jnp.float32),
    )(p, g1, dv, b1, w2)


def _k3_body(p_ref, g_ref, dv_ref, b_ref, bc_ref, wf1_ref, bf1_ref,
             wf2_ref, bf2_ref, o_ref, pool_acc):
    i = pl.program_id(0)

    @pl.when(i == 0)
    def _():
        pool_acc[...] = jnp.zeros_like(pool_acc)

    s = p_ref[0] + p_ref[1] + g_ref[...]
    a = jnp.maximum(s * dv_ref[...] + b_ref[...], 0.0)
    ones = jnp.ones((_BLK, 1), jnp.float32)
    a_aug = jnp.concatenate([a, ones], axis=1)
    onehot = (bc_ref[...] == lax.broadcasted_iota(jnp.int32, (_BLK, 2 * G), 1)
              ).astype(jnp.float32)
    pool_acc[...] += lax.dot_general(
        onehot, a_aug, (((0,), (0,)), ((), ())),
        preferred_element_type=jnp.float32)

    @pl.when(i == _NBLK - 1)
    def _():
        sums = pool_acc[:, :H]
        cnt = jnp.maximum(pool_acc[:, H:H + 1], 1.0)
        mean = sums / cnt
        comb = jnp.concatenate([mean[:G], mean[G:]], axis=1)
        hh = jnp.maximum(
            jnp.dot(comb, wf1_ref[...], preferred_element_type=jnp.float32)
            + bf1_ref[...], 0.0)
        o_ref[...] = (jnp.dot(hh, wf2_ref[...], preferred_element_type=jnp.float32)
                      + bf2_ref[...])


def _k3(p, g2, dv, b2, bc, wf1, bf1, wf2, bf2):
    return pl.pallas_call(
        _k3_body,
        grid=(_NBLK,),
        in_specs=[
            pl.BlockSpec((NC, _BLK, H), lambda i: (0, i, 0)),
            pl.BlockSpec((_BLK, H), lambda i: (i, 0)),
            pl.BlockSpec((_BLK, 1), lambda i: (i, 0)),
            pl.BlockSpec((1, H), lambda i: (0, 0)),
            pl.BlockSpec((_BLK, 1), lambda i: (i, 0)),
            pl.BlockSpec((2 * H, H), lambda i: (0, 0)),
            pl.BlockSpec((1, H), lambda i: (0, 0)),
            pl.BlockSpec((H, 2), lambda i: (0, 0)),
            pl.BlockSpec((1, 2), lambda i: (0, 0)),
        ],
        out_specs=pl.BlockSpec((G, 2), lambda i: (0, 0)),
        out_shape=jax.ShapeDtypeStruct((G, 2), jnp.float32),
        scratch_shapes=[pltpu.VMEM((2 * G, H + 1), jnp.float32)],
    )(p, g2, dv, b2, bc, wf1, bf1, wf2, bf2)


def kernel(x1, edge_index1, batch1, x2, edge_index2, batch2,
           W1, b1, W2, b2, Wf1, bf1, Wf2, bf2):
    x = jnp.concatenate([x1, x2], axis=0)
    x = jnp.pad(x, ((0, NP - NN), (0, 0)))
    src = jnp.concatenate([edge_index1[0], edge_index2[0] + N])
    dst = jnp.concatenate([edge_index1[1], edge_index2[1] + N])
    src = jnp.pad(src, (0, EP - EE), constant_values=NN).reshape(NW, NPH, CPP, CH)
    dst = jnp.pad(dst, (0, EP - EE), constant_values=NN).reshape(NW, NPH, CPP, CH)
    bc = jnp.concatenate([batch1, batch2 + G])
    bc = jnp.pad(bc, (0, NP - NN), constant_values=2 * G).reshape(NP, 1)

    z1 = jnp.zeros((RPT,), jnp.float32)
    zrow = jnp.zeros((RPT, H), jnp.float32)
    ones = jnp.ones((CH,), jnp.float32)

    degp = _sc_degree(dst, z1, ones)
    g1, dv = _k1(x, W1, degp.reshape(NC, NP, 1))
    p1 = _sc_scatter(g1, src, dst, zrow)
    g2 = _k2(p1, g1, dv, b1.reshape(1, H), W2)
    p2 = _sc_scatter(g2, src, dst, zrow)
    out = _k3(p2, g2, dv, b2.reshape(1, H), bc,
              Wf1, bf1.reshape(1, H), Wf2, bf2.reshape(1, 2))
    return out


# speedup vs baseline: 1.2440x; 1.2440x over previous
import functools

import jax
import jax.numpy as jnp
from jax import lax
from jax.experimental import pallas as pl
from jax.experimental.pallas import tpu as pltpu
from jax.experimental.pallas import tpu_sc as plsc

N = 10000
F_IN = 128
H = 64
G = 128
E = 320000

NN = 2 * N
NC, NS = 2, 16
NW = NC * NS
RPT = 1280
NP = NS * RPT
EE = 2 * E
CH = 128
EPW = 20480
EP = NW * EPW
NCH = EPW // CH
NPH = 4
CPP = NCH // NPH

@functools.lru_cache(maxsize=None)
def _sc_mesh():
    return plsc.VectorSubcoreMesh(core_axis_name="c", subcore_axis_name="s",
                                  num_cores=NC, num_subcores=NS)


def _sc_degree_body(dst_hbm, zeros_hbm, ones_hbm, out_hbm,
                    accum, dsts, ones_v, ssem):
    c = lax.axis_index("c")
    s = lax.axis_index("s")
    wid = s * NC + c
    pltpu.sync_copy(zeros_hbm, accum.at[pl.ds(s * RPT, RPT)])
    pltpu.sync_copy(ones_hbm, ones_v)
    pltpu.sync_copy(dst_hbm.at[wid], dsts)
    plsc.subcore_barrier()

    def grp(i, carry):
        base = 8 * i
        for b in range(8):
            pltpu.async_copy(ones_v, accum.at[dsts.at[base + b]], ssem,
                             add=True)
        for b in range(8):
            pltpu.make_async_copy(ones_v, accum.at[dsts.at[base + b]],
                                  ssem).wait()
        return carry

    lax.fori_loop(0, NCH // 8, grp, 0)
    plsc.subcore_barrier()
    pltpu.sync_copy(accum.at[pl.ds(s * RPT, RPT)], out_hbm.at[c, pl.ds(s * RPT, RPT)])


@functools.lru_cache(maxsize=None)
def _sc_degree_kernel():
    return pl.kernel(
        _sc_degree_body,
        out_type=jax.ShapeDtypeStruct((NC, NP), jnp.float32),
        mesh=_sc_mesh(),
        scratch_types=[
            pltpu.VMEM_SHARED((NP,), jnp.float32),
            pltpu.VMEM((NCH, CH), jnp.int32),
            pltpu.VMEM((CH,), jnp.float32),
            pltpu.SemaphoreType.DMA,
        ],
    )


def _sc_degree(dst, z1, ones):
    return _sc_degree_kernel()(dst, z1, ones)


def _sc_scatter_body(g_hbm, sd_hbm, zrow_hbm, out_hbm,
                     accum, idx, rows, isem, gsem, ssem):
    c = lax.axis_index("c")
    s = lax.axis_index("s")
    wid = s * NC + c
    pltpu.sync_copy(zrow_hbm, accum.at[pl.ds(s * RPT, RPT)])
    plsc.subcore_barrier()

    for j in range(4):
        pltpu.async_copy(sd_hbm.at[wid, j], idx.at[j], isem.at[j])
    for j in range(3):
        pltpu.make_async_copy(sd_hbm.at[wid, j], idx.at[j], isem.at[j]).wait()
        pltpu.async_copy(g_hbm.at[idx.at[j, 0]], rows.at[j], gsem.at[j])

    def octet(i, carry):
        for b in range(8):
            cc = 8 * i + b

            @pl.when(cc >= 1)
            def _():
                pltpu.make_async_copy(rows.at[(b + 3) % 4],
                                      accum.at[idx.at[(b + 7) % 8, 1]],
                                      ssem.at[(b + 3) % 4]).wait()

            @pl.when(cc + 4 < NCH)
            def _():
                pltpu.async_copy(sd_hbm.at[wid, cc + 4], idx.at[(b + 4) % 8],
                                 isem.at[(b + 4) % 8])

            @pl.when(cc + 3 < NCH)
            def _():
                pltpu.make_async_copy(sd_hbm.at[wid, cc + 3],
                                      idx.at[(b + 3) % 8],
                                      isem.at[(b + 3) % 8]).wait()
                pltpu.async_copy(g_hbm.at[idx.at[(b + 3) % 8, 0]],
                                 rows.at[(b + 3) % 4], gsem.at[(b + 3) % 4])

            pltpu.make_async_copy(g_hbm.at[idx.at[b % 8, 0]], rows.at[b % 4],
                                  gsem.at[b % 4]).wait()
            pltpu.async_copy(rows.at[b % 4], accum.at[idx.at[b % 8, 1]],
                             ssem.at[b % 4], add=True)
        return carry

    lax.fori_loop(0, NCH // 8, octet, 0)
    pltpu.make_async_copy(rows.at[(NCH - 1) % 4],
                          accum.at[idx.at[(NCH - 1) % 8, 1]],
                          ssem.at[(NCH - 1) % 4]).wait()
    plsc.subcore_barrier()
    pltpu.sync_copy(accum.at[pl.ds(s * RPT, RPT)], out_hbm.at[c, pl.ds(s * RPT, RPT)])


@functools.lru_cache(maxsize=None)
def _sc_scatter_kernel():
    return pl.kernel(
        _sc_scatter_body,
        out_type=jax.ShapeDtypeStruct((NC, NP, H), jnp.float32),
        mesh=_sc_mesh(),
        scratch_types=[
            pltpu.VMEM_SHARED((NP, H), jnp.float32),
            pltpu.VMEM((8, 2, CH), jnp.int32),
            pltpu.VMEM((4, CH, H), jnp.float32),
            pltpu.SemaphoreType.DMA((8,)),
            pltpu.SemaphoreType.DMA((4,)),
            pltpu.SemaphoreType.DMA((4,)),
        ],
        compiler_params=pltpu.CompilerParams(use_tc_tiling_on_sc=False),
    )


def _sc_scatter(g, sd, zrow):
    return _sc_scatter_kernel()(g, sd, zrow)


_BLK = 1280
_NBLK = NP // _BLK


def _k1_body(x_ref, w_ref, dp_ref, g_ref, dv_ref):
    d = dp_ref[0] + dp_ref[1] + 1.0
    dv = lax.rsqrt(d)
    h = jnp.dot(x_ref[...], w_ref[...], preferred_element_type=jnp.float32)
    g_ref[...] = h * dv
    dv_ref[...] = dv


def _k1(x, w1, degp):
    return pl.pallas_call(
        _k1_body,
        grid=(_NBLK,),
        in_specs=[
            pl.BlockSpec((_BLK, F_IN), lambda i: (i, 0)),
            pl.BlockSpec((F_IN, H), lambda i: (0, 0)),
            pl.BlockSpec((NC, _BLK, 1), lambda i: (0, i, 0)),
        ],
        out_specs=[
            pl.BlockSpec((_BLK, H), lambda i: (i, 0)),
            pl.BlockSpec((_BLK, 1), lambda i: (i, 0)),
        ],
        out_shape=[
            jax.ShapeDtypeStruct((NP, H), jnp.float32),
            jax.ShapeDtypeStruct((NP, 1), jnp.float32),
        ],
    )(x, w1, degp)


def _k2_body(p_ref, g_ref, dv_ref, b_ref, w_ref, o_ref):
    s = p_ref[0] + p_ref[1] + g_ref[...]
    a = jnp.maximum(s * dv_ref[...] + b_ref[...], 0.0)
    o_ref[...] = jnp.dot(a, w_ref[...], preferred_element_type=jnp.float32) * dv_ref[...]


def _k2(p, g1, dv, b1, w2):
    return pl.pallas_call(
        _k2_body,
        grid=(_NBLK,),
        in_specs=[
            pl.BlockSpec((NC, _BLK, H), lambda i: (0, i, 0)),
            pl.BlockSpec((_BLK, H), lambda i: (i, 0)),
            pl.BlockSpec((_BLK, 1), lambda i: (i, 0)),
            pl.BlockSpec((1, H), lambda i: (0, 0)),
            pl.BlockSpec((H, H), lambda i: (0, 0)),
        ],
        out_specs=pl.BlockSpec((_BLK, H), lambda i: (i, 0)),
        out_shape=jax.ShapeDtypeStruct((NP, H), jnp.float32),
    )(p, g1, dv, b1, w2)


def _k3_body(p_ref, g_ref, dv_ref, b_ref, bc_ref, wf1_ref, bf1_ref,
             wf2_ref, bf2_ref, o_ref, pool_acc):
    i = pl.program_id(0)

    @pl.when(i == 0)
    def _():
        pool_acc[...] = jnp.zeros_like(pool_acc)

    s = p_ref[0] + p_ref[1] + g_ref[...]
    a = jnp.maximum(s * dv_ref[...] + b_ref[...], 0.0)
    ones = jnp.ones((_BLK, 1), jnp.float32)
    a_aug = jnp.concatenate([a, ones], axis=1)
    onehot = (bc_ref[...] == lax.broadcasted_iota(jnp.int32, (_BLK, 2 * G), 1)
              ).astype(jnp.float32)
    pool_acc[...] += lax.dot_general(
        onehot, a_aug, (((0,), (0,)), ((), ())),
        preferred_element_type=jnp.float32)

    @pl.when(i == _NBLK - 1)
    def _():
        sums = pool_acc[:, :H]
        cnt = jnp.maximum(pool_acc[:, H:H + 1], 1.0)
        mean = sums / cnt
        comb = jnp.concatenate([mean[:G], mean[G:]], axis=1)
        hh = jnp.maximum(
            jnp.dot(comb, wf1_ref[...], preferred_element_type=jnp.float32)
            + bf1_ref[...], 0.0)
        o_ref[...] = (jnp.dot(hh, wf2_ref[...], preferred_element_type=jnp.float32)
                      + bf2_ref[...])


def _k3(p, g2, dv, b2, bc, wf1, bf1, wf2, bf2):
    return pl.pallas_call(
        _k3_body,
        grid=(_NBLK,),
        in_specs=[
            pl.BlockSpec((NC, _BLK, H), lambda i: (0, i, 0)),
            pl.BlockSpec((_BLK, H), lambda i: (i, 0)),
            pl.BlockSpec((_BLK, 1), lambda i: (i, 0)),
            pl.BlockSpec((1, H), lambda i: (0, 0)),
            pl.BlockSpec((_BLK, 1), lambda i: (i, 0)),
            pl.BlockSpec((2 * H, H), lambda i: (0, 0)),
            pl.BlockSpec((1, H), lambda i: (0, 0)),
            pl.BlockSpec((H, 2), lambda i: (0, 0)),
            pl.BlockSpec((1, 2), lambda i: (0, 0)),
        ],
        out_specs=pl.BlockSpec((G, 2), lambda i: (0, 0)),
        out_shape=jax.ShapeDtypeStruct((G, 2), jnp.float32),
        scratch_shapes=[pltpu.VMEM((2 * G, H + 1), jnp.float32)],
    )(p, g2, dv, b2, bc, wf1, bf1, wf2, bf2)


def kernel(x1, edge_index1, batch1, x2, edge_index2, batch2,
           W1, b1, W2, b2, Wf1, bf1, Wf2, bf2):
    x = jnp.concatenate([x1, x2], axis=0)
    x = jnp.pad(x, ((0, NP - NN), (0, 0)))
    src = jnp.concatenate([edge_index1[0], edge_index2[0] + N])
    dst = jnp.concatenate([edge_index1[1], edge_index2[1] + N])
    src = jnp.pad(src, (0, EP - EE), constant_values=NN).reshape(NW, NCH, CH)
    dst = jnp.pad(dst, (0, EP - EE), constant_values=NN).reshape(NW, NCH, CH)
    sd = jnp.stack([src, dst], axis=2)
    bc = jnp.concatenate([batch1, batch2 + G])
    bc = jnp.pad(bc, (0, NP - NN), constant_values=2 * G).reshape(NP, 1)

    z1 = jnp.zeros((RPT,), jnp.float32)
    zrow = jnp.zeros((RPT, H), jnp.float32)
    ones = jnp.ones((CH,), jnp.float32)

    degp = _sc_degree(dst, z1, ones)
    g1, dv = _k1(x, W1, degp.reshape(NC, NP, 1))
    p1 = _sc_scatter(g1, sd, zrow)
    g2 = _k2(p1, g1, dv, b1.reshape(1, H), W2)
    p2 = _sc_scatter(g2, sd, zrow)
    out = _k3(p2, g2, dv, b2.reshape(1, H), bc,
              Wf1, bf1.reshape(1, H), Wf2, bf2.reshape(1, 2))
    return out


# speedup vs baseline: 1.2617x; 1.0143x over previous
import functools

import jax
import jax.numpy as jnp
from jax import lax
from jax.experimental import pallas as pl
from jax.experimental.pallas import tpu as pltpu
from jax.experimental.pallas import tpu_sc as plsc

N = 10000
F_IN = 128
H = 64
G = 128
E = 320000

NN = 2 * N
NC, NS = 2, 16
NW = NC * NS
RPT = 1280
NP = NS * RPT
EE = 2 * E
CH = 128
EPW = 20480
EP = NW * EPW
NCH = EPW // CH
TOTCH = NW * NCH
K0 = 256
K1 = 64

@functools.lru_cache(maxsize=None)
def _sc_mesh():
    return plsc.VectorSubcoreMesh(core_axis_name="c", subcore_axis_name="s",
                                  num_cores=NC, num_subcores=NS)


def _sc_degree_body(dst_hbm, zeros_hbm, ones_hbm, out_hbm,
                    accum, dsts, ones_v, ssem):
    c = lax.axis_index("c")
    s = lax.axis_index("s")
    wid = s * NC + c
    pltpu.sync_copy(zeros_hbm, accum.at[pl.ds(s * RPT, RPT)])
    pltpu.sync_copy(ones_hbm, ones_v)
    pltpu.sync_copy(dst_hbm.at[wid], dsts)
    plsc.subcore_barrier()

    def grp(i, carry):
        base = 8 * i
        for b in range(8):
            pltpu.async_copy(ones_v, accum.at[dsts.at[base + b]], ssem,
                             add=True)
        for b in range(8):
            pltpu.make_async_copy(ones_v, accum.at[dsts.at[base + b]],
                                  ssem).wait()
        return carry

    lax.fori_loop(0, NCH // 8, grp, 0)
    plsc.subcore_barrier()
    pltpu.sync_copy(accum.at[pl.ds(s * RPT, RPT)], out_hbm.at[c, pl.ds(s * RPT, RPT)])


@functools.lru_cache(maxsize=None)
def _sc_degree_kernel():
    return pl.kernel(
        _sc_degree_body,
        out_type=jax.ShapeDtypeStruct((NC, NP), jnp.float32),
        mesh=_sc_mesh(),
        scratch_types=[
            pltpu.VMEM_SHARED((NP,), jnp.float32),
            pltpu.VMEM((NCH, CH), jnp.int32),
            pltpu.VMEM((CH,), jnp.float32),
            pltpu.SemaphoreType.DMA,
        ],
    )


def _sc_degree(dst, z1, ones):
    return _sc_degree_kernel()(dst, z1, ones)


def _sc_scatter_body(g_hbm, sd_hbm, zrow_hbm, out_hbm,
                     accum, idx, rows, isem, gsem, ssem):
    c = lax.axis_index("c")
    s = lax.axis_index("s")
    base = jnp.where(c == 0, s * K0, NS * K0 + s * K1)
    k = jnp.where(c == 0, K0, K1)
    pltpu.sync_copy(zrow_hbm, accum.at[pl.ds(s * RPT, RPT)])
    plsc.subcore_barrier()

    for j in range(4):
        pltpu.async_copy(sd_hbm.at[base + j], idx.at[j], isem.at[j])
    for j in range(3):
        pltpu.make_async_copy(sd_hbm.at[base + j], idx.at[j], isem.at[j]).wait()
        pltpu.async_copy(g_hbm.at[idx.at[j, 0]], rows.at[j], gsem.at[j])

    def octet(i, carry):
        for b in range(8):
            cc = 8 * i + b

            @pl.when(cc >= 1)
            def _():
                pltpu.make_async_copy(rows.at[(b + 3) % 4],
                                      accum.at[idx.at[(b + 7) % 8, 1]],
                                      ssem.at[(b + 3) % 4]).wait()

            @pl.when(cc + 4 < k)
            def _():
                pltpu.async_copy(sd_hbm.at[base + cc + 4], idx.at[(b + 4) % 8],
                                 isem.at[(b + 4) % 8])

            @pl.when(cc + 3 < k)
            def _():
                pltpu.make_async_copy(sd_hbm.at[base + cc + 3],
                                      idx.at[(b + 3) % 8],
                                      isem.at[(b + 3) % 8]).wait()
                pltpu.async_copy(g_hbm.at[idx.at[(b + 3) % 8, 0]],
                                 rows.at[(b + 3) % 4], gsem.at[(b + 3) % 4])

            pltpu.make_async_copy(g_hbm.at[idx.at[b % 8, 0]], rows.at[b % 4],
                                  gsem.at[b % 4]).wait()
            pltpu.async_copy(rows.at[b % 4], accum.at[idx.at[b % 8, 1]],
                             ssem.at[b % 4], add=True)
        return carry

    lax.fori_loop(0, k // 8, octet, 0)
    pltpu.make_async_copy(rows.at[3], accum.at[idx.at[7, 1]],
                          ssem.at[3]).wait()
    plsc.subcore_barrier()
    pltpu.sync_copy(accum.at[pl.ds(s * RPT, RPT)], out_hbm.at[c, pl.ds(s * RPT, RPT)])


@functools.lru_cache(maxsize=None)
def _sc_scatter_kernel():
    return pl.kernel(
        _sc_scatter_body,
        out_type=jax.ShapeDtypeStruct((NC, NP, H), jnp.float32),
        mesh=_sc_mesh(),
        scratch_types=[
            pltpu.VMEM_SHARED((NP, H), jnp.float32),
            pltpu.VMEM((8, 2, CH), jnp.int32),
            pltpu.VMEM((4, CH, H), jnp.float32),
            pltpu.SemaphoreType.DMA((8,)),
            pltpu.SemaphoreType.DMA((4,)),
            pltpu.SemaphoreType.DMA((4,)),
        ],
        compiler_params=pltpu.CompilerParams(use_tc_tiling_on_sc=False),
    )


def _sc_scatter(g, sd, zrow):
    return _sc_scatter_kernel()(g, sd, zrow)


_BLK = 1280
_NBLK = NP // _BLK


def _k1_body(x_ref, w_ref, dp_ref, g_ref, dv_ref):
    d = dp_ref[0] + dp_ref[1] + 1.0
    dv = lax.rsqrt(d)
    h = jnp.dot(x_ref[...], w_ref[...], preferred_element_type=jnp.float32)
    g_ref[...] = h * dv
    dv_ref[...] = dv


def _k1(x, w1, degp):
    return pl.pallas_call(
        _k1_body,
        grid=(_NBLK,),
        in_specs=[
            pl.BlockSpec((_BLK, F_IN), lambda i: (i, 0)),
            pl.BlockSpec((F_IN, H), lambda i: (0, 0)),
            pl.BlockSpec((NC, _BLK, 1), lambda i: (0, i, 0)),
        ],
        out_specs=[
            pl.BlockSpec((_BLK, H), lambda i: (i, 0)),
            pl.BlockSpec((_BLK, 1), lambda i: (i, 0)),
        ],
        out_shape=[
            jax.ShapeDtypeStruct((NP, H), jnp.float32),
            jax.ShapeDtypeStruct((NP, 1), jnp.float32),
        ],
    )(x, w1, degp)


def _k2_body(p_ref, g_ref, dv_ref, b_ref, w_ref, o_ref):
    s = p_ref[0] + p_ref[1] + g_ref[...]
    a = jnp.maximum(s * dv_ref[...] + b_ref[...], 0.0)
    o_ref[...] = jnp.dot(a, w_ref[...], preferred_element_type=jnp.float32) * dv_ref[...]


def _k2(p, g1, dv, b1, w2):
    return pl.pallas_call(
        _k2_body,
        grid=(_NBLK,),
        in_specs=[
            pl.BlockSpec((NC, _BLK, H), lambda i: (0, i, 0)),
            pl.BlockSpec((_BLK, H), lambda i: (i, 0)),
            pl.BlockSpec((_BLK, 1), lambda i: (i, 0)),
            pl.BlockSpec((1, H), lambda i: (0, 0)),
            pl.BlockSpec((H, H), lambda i: (0, 0)),
        ],
        out_specs=pl.BlockSpec((_BLK, H), lambda i: (i, 0)),
        out_shape=jax.ShapeDtypeStruct((NP, H), jnp.float32),
    )(p, g1, dv, b1, w2)


def _k3_body(p_ref, g_ref, dv_ref, b_ref, bc_ref, wf1_ref, bf1_ref,
             wf2_ref, bf2_ref, o_ref, pool_acc):
    i = pl.program_id(0)

    @pl.when(i == 0)
    def _():
        pool_acc[...] = jnp.zeros_like(pool_acc)

    s = p_ref[0] + p_ref[1] + g_ref[...]
    a = jnp.maximum(s * dv_ref[...] + b_ref[...], 0.0)
    ones = jnp.ones((_BLK, 1), jnp.float32)
    a_aug = jnp.concatenate([a, ones], axis=1)
    onehot = (bc_ref[...] == lax.broadcasted_iota(jnp.int32, (_BLK, 2 * G), 1)
              ).astype(jnp.float32)
    pool_acc[...] += lax.dot_general(
        onehot, a_aug, (((0,), (0,)), ((), ())),
        preferred_element_type=jnp.float32)

    @pl.when(i == _NBLK - 1)
    def _():
        sums = pool_acc[:, :H]
        cnt = jnp.maximum(pool_acc[:, H:H + 1], 1.0)
        mean = sums / cnt
        comb = jnp.concatenate([mean[:G], mean[G:]], axis=1)
        hh = jnp.maximum(
            jnp.dot(comb, wf1_ref[...], preferred_element_type=jnp.float32)
            + bf1_ref[...], 0.0)
        o_ref[...] = (jnp.dot(hh, wf2_ref[...], preferred_element_type=jnp.float32)
                      + bf2_ref[...])


def _k3(p, g2, dv, b2, bc, wf1, bf1, wf2, bf2):
    return pl.pallas_call(
        _k3_body,
        grid=(_NBLK,),
        in_specs=[
            pl.BlockSpec((NC, _BLK, H), lambda i: (0, i, 0)),
            pl.BlockSpec((_BLK, H), lambda i: (i, 0)),
            pl.BlockSpec((_BLK, 1), lambda i: (i, 0)),
            pl.BlockSpec((1, H), lambda i: (0, 0)),
            pl.BlockSpec((_BLK, 1), lambda i: (i, 0)),
            pl.BlockSpec((2 * H, H), lambda i: (0, 0)),
            pl.BlockSpec((1, H), lambda i: (0, 0)),
            pl.BlockSpec((H, 2), lambda i: (0, 0)),
            pl.BlockSpec((1, 2), lambda i: (0, 0)),
        ],
        out_specs=pl.BlockSpec((G, 2), lambda i: (0, 0)),
        out_shape=jax.ShapeDtypeStruct((G, 2), jnp.float32),
        scratch_shapes=[pltpu.VMEM((2 * G, H + 1), jnp.float32)],
    )(p, g2, dv, b2, bc, wf1, bf1, wf2, bf2)


def kernel(x1, edge_index1, batch1, x2, edge_index2, batch2,
           W1, b1, W2, b2, Wf1, bf1, Wf2, bf2):
    x = jnp.concatenate([x1, x2], axis=0)
    x = jnp.pad(x, ((0, NP - NN), (0, 0)))
    src = jnp.concatenate([edge_index1[0], edge_index2[0] + N])
    dst = jnp.concatenate([edge_index1[1], edge_index2[1] + N])
    src = jnp.pad(src, (0, EP - EE), constant_values=NN).reshape(NW, NCH, CH)
    dst = jnp.pad(dst, (0, EP - EE), constant_values=NN).reshape(NW, NCH, CH)
    sd = jnp.stack([src, dst], axis=2).reshape(TOTCH, 2, CH)
    bc = jnp.concatenate([batch1, batch2 + G])
    bc = jnp.pad(bc, (0, NP - NN), constant_values=2 * G).reshape(NP, 1)

    z1 = jnp.zeros((RPT,), jnp.float32)
    zrow = jnp.zeros((RPT, H), jnp.float32)
    ones = jnp.ones((CH,), jnp.float32)

    degp = _sc_degree(dst, z1, ones)
    g1, dv = _k1(x, W1, degp.reshape(NC, NP, 1))
    p1 = _sc_scatter(g1, sd, zrow)
    g2 = _k2(p1, g1, dv, b1.reshape(1, H), W2)
    p2 = _sc_scatter(g2, sd, zrow)
    out = _k3(p2, g2, dv, b2.reshape(1, H), bc,
              Wf1, bf1.reshape(1, H), Wf2, bf2.reshape(1, 2))
    return out
